# baseline (speedup 1.0000x reference)
import jax
import jax.numpy as jnp
from jax import lax
from jax.experimental import pallas as pl
from jax.experimental.pallas import tpu as pltpu
from jax.experimental.pallas import tpu_sc as plsc

B, S, D = 4096, 50, 100
DP = 128
NC, NS = 2, 16
NW = NC * NS
BPW = B // NW
NB = 8
CHUNKS = BPW // NB
SP = 56
VB = 8192


def _tp_body(in_ref, out_ref):
    t = jnp.transpose(in_ref[...], (1, 0))
    out_ref[...] = jnp.pad(t, ((0, 0), (0, DP - D)))


def _transpose_pad(table_t):
    v = table_t.shape[1]
    return pl.pallas_call(
        _tp_body,
        grid=(pl.cdiv(v, VB),),
        in_specs=[pl.BlockSpec((D, VB), lambda i: (0, i))],
        out_specs=pl.BlockSpec((VB, DP), lambda i: (i, 0)),
        out_shape=jax.ShapeDtypeStruct((v, DP), jnp.float32),
        compiler_params=pltpu.CompilerParams(
            dimension_semantics=("parallel",)),
    )(table_t)


def _ix_body(in_ref, out_ref):
    t = jnp.pad(jnp.transpose(in_ref[...], (1, 0)), ((0, 0), (0, 128 - S)))
    rows = lax.broadcasted_iota(jnp.int32, (B, 128), 0)
    cols = lax.broadcasted_iota(jnp.int32, (B, 128), 1)
    junk = (rows * 977 + cols * 131) % 99991
    out_ref[...] = jnp.where(cols < S, t, junk)


def _idx_relayout(idx_t):
    return pl.pallas_call(
        _ix_body,
        out_shape=jax.ShapeDtypeStruct((B, 128), jnp.int32),
    )(idx_t)


def _sc_body(idx, table, out_hbm, idx_v, rows_v, isem, gsem, osem):
    wid = lax.axis_index("s") * NC + lax.axis_index("c")
    wb0 = wid * BPW

    pltpu.async_copy(idx.at[pl.ds(wb0, BPW), pl.ds(0, SP)],
                     idx_v, isem).wait()

    @pl.loop(0, CHUNKS)
    def _(ci):
        buf = lax.rem(ci, 2)
        b0 = wb0 + ci * NB

        @pl.when(ci >= 2)
        def _():
            pltpu.make_async_copy(
                rows_v.at[buf], out_hbm.at[pl.ds(b0, NB)],
                osem.at[buf]).wait()

        @pl.loop(0, NB)
        def _(j):
            pltpu.async_copy(
                table.at[idx_v.at[ci * NB + j]], rows_v.at[buf, j], gsem)

        @pl.loop(0, NB)
        def _(j):
            pltpu.make_async_copy(
                table.at[idx_v.at[ci * NB + j]], rows_v.at[buf, j],
                gsem).wait()

        pltpu.async_copy(
            rows_v.at[buf], out_hbm.at[pl.ds(b0, NB)], osem.at[buf])

    for buf in (0, 1):
        pltpu.make_async_copy(
            rows_v.at[buf], out_hbm.at[pl.ds(wb0, NB)], osem.at[buf]).wait()


def _sc_gather(idx, table):
    mesh = plsc.VectorSubcoreMesh(core_axis_name="c", subcore_axis_name="s")
    run = pl.kernel(
        _sc_body,
        out_type=jax.ShapeDtypeStruct((B, SP, DP), jnp.float32),
        mesh=mesh,
        scratch_types=[
            pltpu.VMEM((BPW, SP), jnp.int32),
            pltpu.VMEM((2, NB, SP, DP), jnp.float32),
            pltpu.SemaphoreType.DMA,
            pltpu.SemaphoreType.DMA,
            pltpu.SemaphoreType.DMA((2,)),
        ],
        compiler_params=pltpu.CompilerParams(use_tc_tiling_on_sc=False),
    )
    return run(idx, table)


def _asm_body(g0, g1, g2, out_ref):
    for t, g in enumerate((g0, g1, g2)):
        out_ref[t, 0] = jnp.transpose(g[...], (1, 0))[:D]


def _assemble(g0, g1, g2):
    return pl.pallas_call(
        _asm_body,
        grid=(S,),
        in_specs=[pl.BlockSpec((B, DP), lambda s: (0, s)) for _ in range(3)],
        out_specs=pl.BlockSpec((3, 1, D, B), lambda s: (0, s, 0, 0)),
        out_shape=jax.ShapeDtypeStruct((3, S, D, B), jnp.float32),
        compiler_params=pltpu.CompilerParams(
            dimension_semantics=("arbitrary",)),
    )(g0, g1, g2)


def kernel(words_idx, wv_idx, glove_idx, emb_table, cbow_table, glove_table):
    gs = []
    for i, t in ((words_idx, emb_table), (wv_idx, cbow_table),
                 (glove_idx, glove_table)):
        ix = _idx_relayout(jnp.transpose(i.astype(jnp.int32)))
        tp = _transpose_pad(jnp.transpose(t))
        g = _sc_gather(ix, tp)
        gs.append(jnp.reshape(g, (B, SP * DP)))
    y = _assemble(*gs)
    return jnp.transpose(y, (3, 0, 1, 2))


# speedup vs baseline: 1.2027x; 1.2027x over previous
import jax
import jax.numpy as jnp
from jax import lax
from jax.experimental import pallas as pl
from jax.experimental.pallas import tpu as pltpu
from jax.experimental.pallas import tpu_sc as plsc

B, S, D = 4096, 50, 100
DP = 128
NC, NS = 2, 16
NW = NC * NS
BPW = B // NW
NB = 8
CHUNKS = BPW // NB
SP = 56
VB = 16384


def _tp_body(in_ref, out_ref):
    t = jnp.transpose(in_ref[...], (1, 0))
    out_ref[...] = jnp.pad(t, ((0, 0), (0, DP - D)))


def _transpose_pad(table_t):
    v = table_t.shape[1]
    grid = pl.cdiv(v, VB)
    return pl.pallas_call(
        _tp_body,
        grid=(grid,),
        in_specs=[pl.BlockSpec((D, VB), lambda i: (0, i))],
        out_specs=pl.BlockSpec((VB, DP), lambda i: (i, 0)),
        out_shape=jax.ShapeDtypeStruct((v, DP), jnp.float32),
        compiler_params=pltpu.CompilerParams(
            dimension_semantics=("parallel",)),
    )(table_t)


def _ix_body(in_ref, out_ref):
    t = jnp.pad(jnp.transpose(in_ref[...], (1, 0)), ((0, 0), (0, 128 - S)))
    rows = lax.broadcasted_iota(jnp.int32, (B, 128), 0)
    cols = lax.broadcasted_iota(jnp.int32, (B, 128), 1)
    junk = (rows * 977 + cols * 131) % 99991
    out_ref[...] = jnp.where(cols < S, t, junk)


def _idx_relayout(idx_t):
    return pl.pallas_call(
        _ix_body,
        out_shape=jax.ShapeDtypeStruct((B, 128), jnp.int32),
    )(idx_t)


def _sc_body(w_idx, wv_idx, gl_idx, emb_t, cbow_t, gl_t, out_hbm,
             idx_v, rows_v, isem, gsem, osem):
    wid = lax.axis_index("s") * NC + lax.axis_index("c")
    wb0 = wid * BPW

    for t, (idx, table) in enumerate(((w_idx, emb_t), (wv_idx, cbow_t),
                                      (gl_idx, gl_t))):
        pltpu.async_copy(idx.at[pl.ds(wb0, BPW), pl.ds(0, SP)],
                         idx_v, isem).wait()

        @pl.loop(0, CHUNKS)
        def _(ci, t=t, table=table):
            buf = lax.rem(ci, 2)
            b0 = wb0 + ci * NB

            @pl.when(ci >= 2)
            def _():
                pltpu.make_async_copy(
                    rows_v.at[buf],
                    out_hbm.at[pl.ds(b0, NB), t],
                    osem.at[buf]).wait()

            @pl.loop(0, NB)
            def _(j):
                pltpu.async_copy(
                    table.at[idx_v.at[ci * NB + j]],
                    rows_v.at[buf, j],
                    gsem)

            @pl.loop(0, NB)
            def _(j):
                pltpu.make_async_copy(
                    table.at[idx_v.at[ci * NB + j]],
                    rows_v.at[buf, j],
                    gsem).wait()

            pltpu.async_copy(
                rows_v.at[buf],
                out_hbm.at[pl.ds(b0, NB), t],
                osem.at[buf])

        for buf in (0, 1):
            pltpu.make_async_copy(
                rows_v.at[buf],
                out_hbm.at[pl.ds(wb0, NB), t],
                osem.at[buf]).wait()


@jax.jit
def _sc_embed(w_idx, wv_idx, gl_idx, emb_t, cbow_t, gl_t):
    mesh = plsc.VectorSubcoreMesh(core_axis_name="c", subcore_axis_name="s")
    run = pl.kernel(
        _sc_body,
        out_type=jax.ShapeDtypeStruct((B, 3, SP, DP), jnp.float32),
        mesh=mesh,
        scratch_types=[
            pltpu.VMEM((BPW, SP), jnp.int32),
            pltpu.VMEM((2, NB, SP, DP), jnp.float32),
            pltpu.SemaphoreType.DMA,
            pltpu.SemaphoreType.DMA,
            pltpu.SemaphoreType.DMA((2,)),
        ],
        compiler_params=pltpu.CompilerParams(use_tc_tiling_on_sc=False),
    )
    return run(w_idx, wv_idx, gl_idx, emb_t, cbow_t, gl_t)


def kernel(words_idx, wv_idx, glove_idx, emb_table, cbow_table, glove_table):
    idxs = [_idx_relayout(jnp.transpose(i.astype(jnp.int32)))
            for i in (words_idx, wv_idx, glove_idx)]
    tables = [_transpose_pad(jnp.transpose(t))
              for t in (emb_table, cbow_table, glove_table)]
    out = _sc_embed(*idxs, *tables)
    return out[:, :, :S, :D]


# speedup vs baseline: 1.2090x; 1.0052x over previous
import jax
import jax.numpy as jnp
from jax import lax
from jax.experimental import pallas as pl
from jax.experimental.pallas import tpu as pltpu
from jax.experimental.pallas import tpu_sc as plsc

B, S, D = 4096, 50, 100
DP = 128
NC, NS = 2, 16
NW = NC * NS
BPW = B // NW
NB = 8
CHUNKS = BPW // NB
SP = 56
VB = 24576


def _tp_body(in_ref, out_ref):
    t = jnp.transpose(in_ref[...], (1, 0))
    out_ref[...] = jnp.pad(t, ((0, 0), (0, DP - D)))


def _transpose_pad(table_t):
    v = table_t.shape[1]
    grid = pl.cdiv(v, VB)
    return pl.pallas_call(
        _tp_body,
        grid=(grid,),
        in_specs=[pl.BlockSpec((D, VB), lambda i: (0, i))],
        out_specs=pl.BlockSpec((VB, DP), lambda i: (i, 0)),
        out_shape=jax.ShapeDtypeStruct((v, DP), jnp.float32),
        compiler_params=pltpu.CompilerParams(
            dimension_semantics=("parallel",)),
    )(table_t)


def _ix_body(in_ref, out_ref):
    t = jnp.pad(jnp.transpose(in_ref[...], (1, 0)), ((0, 0), (0, 128 - S)))
    rows = lax.broadcasted_iota(jnp.int32, (B, 128), 0)
    cols = lax.broadcasted_iota(jnp.int32, (B, 128), 1)
    junk = (rows * 977 + cols * 131) % 99991
    out_ref[...] = jnp.where(cols < S, t, junk)


def _idx_relayout(idx_t):
    return pl.pallas_call(
        _ix_body,
        out_shape=jax.ShapeDtypeStruct((B, 128), jnp.int32),
    )(idx_t)


def _sc_body(w_idx, wv_idx, gl_idx, emb_t, cbow_t, gl_t, out_hbm,
             idx_v, rows_v, isem, gsem, osem):
    wid = lax.axis_index("s") * NC + lax.axis_index("c")
    wb0 = wid * BPW

    for t, (idx, table) in enumerate(((w_idx, emb_t), (wv_idx, cbow_t),
                                      (gl_idx, gl_t))):
        pltpu.async_copy(idx.at[pl.ds(wb0, BPW), pl.ds(0, SP)],
                         idx_v, isem).wait()

        @pl.loop(0, CHUNKS)
        def _(ci, t=t, table=table):
            buf = lax.rem(ci, 2)
            b0 = wb0 + ci * NB

            @pl.when(ci >= 2)
            def _():
                pltpu.make_async_copy(
                    rows_v.at[buf],
                    out_hbm.at[pl.ds(b0, NB), t],
                    osem.at[buf]).wait()

            @pl.loop(0, NB)
            def _(j):
                pltpu.async_copy(
                    table.at[idx_v.at[ci * NB + j]],
                    rows_v.at[buf, j],
                    gsem)

            @pl.loop(0, NB)
            def _(j):
                pltpu.make_async_copy(
                    table.at[idx_v.at[ci * NB + j]],
                    rows_v.at[buf, j],
                    gsem).wait()

            pltpu.async_copy(
                rows_v.at[buf],
                out_hbm.at[pl.ds(b0, NB), t],
                osem.at[buf])

        for buf in (0, 1):
            pltpu.make_async_copy(
                rows_v.at[buf],
                out_hbm.at[pl.ds(wb0, NB), t],
                osem.at[buf]).wait()


@jax.jit
def _sc_embed(w_idx, wv_idx, gl_idx, emb_t, cbow_t, gl_t):
    mesh = plsc.VectorSubcoreMesh(core_axis_name="c", subcore_axis_name="s")
    run = pl.kernel(
        _sc_body,
        out_type=jax.ShapeDtypeStruct((B, 3, SP, DP), jnp.float32),
        mesh=mesh,
        scratch_types=[
            pltpu.VMEM((BPW, SP), jnp.int32),
            pltpu.VMEM((2, NB, SP, DP), jnp.float32),
            pltpu.SemaphoreType.DMA,
            pltpu.SemaphoreType.DMA,
            pltpu.SemaphoreType.DMA((2,)),
        ],
        compiler_params=pltpu.CompilerParams(use_tc_tiling_on_sc=False),
    )
    return run(w_idx, wv_idx, gl_idx, emb_t, cbow_t, gl_t)


def kernel(words_idx, wv_idx, glove_idx, emb_table, cbow_table, glove_table):
    idxs = [_idx_relayout(jnp.transpose(i.astype(jnp.int32)))
            for i in (words_idx, wv_idx, glove_idx)]
    tables = [_transpose_pad(jnp.transpose(t))
              for t in (emb_table, cbow_table, glove_table)]
    out = _sc_embed(*idxs, *tables)
    return out[:, :, :S, :D]
